# baseline (speedup 1.0000x reference)
import functools

import jax
import jax.numpy as jnp
import numpy as np
from jax import lax
from jax.experimental import pallas as pl
from jax.experimental.pallas import tpu as pltpu
from jax.experimental.pallas import tpu_sc as plsc

N = 10000
D = 128
H = 128
C = 2
E = 320000

NC = 2
NS = 16
NT = NC * NS
CH = 128
CPT = 80
BLK = 16
EP = NT * CPT * CH
NP = 10240
RPT = NP // NS
RPB = RPT // CH

_F32 = jnp.float32


def _sc_mesh():
    return plsc.VectorSubcoreMesh(
        core_axis_name="c", subcore_axis_name="s",
        num_cores=NC, num_subcores=NS)


def _make_deg():
    @functools.partial(
        pl.kernel,
        out_type=[jax.ShapeDtypeStruct((NP,), _F32),
                  jax.ShapeDtypeStruct((NP,), _F32)],
        mesh=_sc_mesh(),
        scratch_types=[
            pltpu.VMEM((CPT, CH), jnp.int32),
            pltpu.VMEM((CH,), _F32),
            pltpu.VMEM((RPT,), _F32),
            pltpu.VMEM_SHARED((NP,), _F32),
        ],
    )
    def deg_kernel(dst_hbm, out0_hbm, out1_hbm, dst_v, ones_v, deg_v, acc):
        cid = lax.axis_index("c")
        sid = lax.axis_index("s")
        wid = cid * NS + sid
        r0 = sid * RPT

        for k in range(CH // 16):
            ones_v[pl.ds(k * 16, 16)] = jnp.ones((16,), _F32)

        def zb(i, carry):
            deg_v[pl.ds(i * 16, 16)] = jnp.zeros((16,), _F32)
            return carry

        lax.fori_loop(0, RPT // 16, zb, 0)
        pltpu.sync_copy(deg_v, acc.at[pl.ds(r0, RPT)])
        pltpu.sync_copy(dst_hbm.at[wid], dst_v)
        plsc.subcore_barrier()

        def body(j, carry):
            pltpu.sync_copy(ones_v, acc.at[dst_v.at[j]], add=True)
            return carry

        lax.fori_loop(0, CPT, body, 0)
        plsc.subcore_barrier()
        pltpu.sync_copy(acc.at[pl.ds(r0, RPT)], deg_v)

        @pl.when(cid == 0)
        def _():
            pltpu.sync_copy(deg_v, out0_hbm.at[pl.ds(r0, RPT)])

        @pl.when(cid != 0)
        def _():
            pltpu.sync_copy(deg_v, out1_hbm.at[pl.ds(r0, RPT)])

    return deg_kernel


def _make_agg(F, tc_tiling=True):
    @functools.partial(
        pl.kernel,
        out_type=jax.ShapeDtypeStruct((NC, NP, F), _F32),
        mesh=_sc_mesh(),
        compiler_params=(None if tc_tiling else
                         pltpu.CompilerParams(use_tc_tiling_on_sc=False)),
        scratch_types=[
            pltpu.VMEM((BLK, CH), jnp.int32),
            pltpu.VMEM((BLK, CH), jnp.int32),
            pltpu.VMEM((CH, F), _F32),
            pltpu.VMEM((CH, F), _F32),
            pltpu.VMEM_SHARED((NP, F), _F32),
            pltpu.SemaphoreType.DMA,
            pltpu.SemaphoreType.DMA,
        ],
    )
    def agg_kernel(y_hbm, zeros_hbm, src_hbm, dst_hbm, out_hbm,
                   src_v, dst_v, buf_a, buf_b, acc, sem_a, sem_b):
        cid = lax.axis_index("c")
        sid = lax.axis_index("s")
        wid = cid * NS + sid
        r0 = sid * RPT

        pltpu.sync_copy(zeros_hbm, buf_a)

        def ib(i, carry):
            pltpu.sync_copy(buf_a, acc.at[pl.ds(r0 + i * CH, CH)])
            return carry

        lax.fori_loop(0, RPB, ib, 0)
        plsc.subcore_barrier()

        def blk(b, carry):
            pltpu.sync_copy(src_hbm.at[wid, pl.ds(b * BLK, BLK)], src_v)
            pltpu.sync_copy(dst_hbm.at[wid, pl.ds(b * BLK, BLK)], dst_v)
            pltpu.async_copy(y_hbm.at[src_v.at[0]], buf_a, sem_a)

            def body(j, carry2):
                ca = 2 * j
                cb = 2 * j + 1
                pltpu.async_copy(y_hbm.at[src_v.at[cb]], buf_b, sem_b)
                pltpu.make_async_copy(
                    y_hbm.at[src_v.at[ca]], buf_a, sem_a).wait()
                pltpu.sync_copy(buf_a, acc.at[dst_v.at[ca]], add=True)
                nxt = lax.rem(ca + 2, BLK)
                pltpu.async_copy(y_hbm.at[src_v.at[nxt]], buf_a, sem_a)
                pltpu.make_async_copy(
                    y_hbm.at[src_v.at[cb]], buf_b, sem_b).wait()
                pltpu.sync_copy(buf_b, acc.at[dst_v.at[cb]], add=True)
                return carry2

            lax.fori_loop(0, BLK // 2, body, 0)
            pltpu.make_async_copy(y_hbm.at[src_v.at[0]], buf_a, sem_a).wait()
            return carry

        lax.fori_loop(0, CPT // BLK, blk, 0)
        plsc.subcore_barrier()

        def ob(i, carry):
            pltpu.sync_copy(acc.at[pl.ds(r0 + i * CH, CH)], buf_a)
            pltpu.sync_copy(buf_a, out_hbm.at[cid, pl.ds(r0 + i * CH, CH)])
            return carry

        lax.fori_loop(0, RPB, ob, 0)

    return agg_kernel


@functools.cache
def _sc_calls():
    return _make_deg(), _make_agg(H), _make_agg(16, tc_tiling=False)


def _tc_mm_body(x_ref, w_ref, hw_ref):
    hw_ref[...] = jnp.dot(x_ref[...], w_ref[...], preferred_element_type=_F32)


def _tc_mm(x, w1):
    return pl.pallas_call(
        _tc_mm_body,
        out_shape=jax.ShapeDtypeStruct((N, H), _F32),
    )(x, w1)


def _tc_prep_body(hw_ref, d0_ref, d1_ref, y_ref, dinv_ref):
    deg = d0_ref[...] + d1_ref[...] + 1.0
    dinv = lax.rsqrt(deg)
    dinv_ref[...] = dinv
    y_ref[0:N, :] = hw_ref[...] * dinv[0:N, None]
    y_ref[N:NP, :] = jnp.zeros((NP - N, H), _F32)


def _tc_prep(hw, d0, d1):
    return pl.pallas_call(
        _tc_prep_body,
        out_shape=[
            jax.ShapeDtypeStruct((NP, H), _F32),
            jax.ShapeDtypeStruct((NP,), _F32),
        ],
    )(hw, d0, d1)


def _tc_mid_body(ys_ref, a_ref, dinv_ref, b_ref, g_ref, be_ref,
                 w_ref, y_ref):
    dinv = dinv_ref[...]
    agg = ys_ref[0:N, :] + a_ref[0, 0:N, :] + a_ref[1, 0:N, :]
    t = (agg * dinv[0:N, None]) + b_ref[...]
    mu = jnp.mean(t, axis=0)
    var = jnp.mean((t - mu) ** 2, axis=0)
    hbn = (t - mu) * lax.rsqrt(var + 1e-5) * g_ref[...] + be_ref[...]
    hbn = jnp.maximum(hbn, 0.0)
    hw = jnp.dot(hbn, w_ref[...], preferred_element_type=_F32)
    y_ref[0:N, :] = hw * dinv[0:N, None]
    y_ref[N:NP, :] = jnp.zeros((NP - N, w_ref.shape[1]), _F32)


def _tc_mid(ys, a, dinv, b, g, be, w):
    fout = w.shape[1]
    return pl.pallas_call(
        _tc_mid_body,
        out_shape=jax.ShapeDtypeStruct((NP, fout), _F32),
    )(ys, a, dinv, b, g, be, w)


def _tc_last_body(ys_ref, a_ref, dinv_ref, b_ref, g_ref, be_ref, w_ref,
                  y_ref):
    dinv = dinv_ref[...]
    agg = ys_ref[0:N, :] + a_ref[0, 0:N, :] + a_ref[1, 0:N, :]
    t = (agg * dinv[0:N, None]) + b_ref[...]
    mu = jnp.mean(t, axis=0)
    var = jnp.mean((t - mu) ** 2, axis=0)
    hbn = (t - mu) * lax.rsqrt(var + 1e-5) * g_ref[...] + be_ref[...]
    hbn = jnp.maximum(hbn, 0.0)
    hw = jnp.dot(hbn, w_ref[...], preferred_element_type=_F32)
    y_ref[0:N, :] = hw * dinv[0:N, None]
    y_ref[N:NP, :] = jnp.zeros((NP - N, 16), _F32)


def _tc_last(ys, a, dinv, b, g, be, w):
    return pl.pallas_call(
        _tc_last_body,
        out_shape=jax.ShapeDtypeStruct((NP, 16), _F32),
    )(ys, a, dinv, b, g, be, w)


def _tc_fin_body(ys_ref, a_ref, dinv_ref, b_ref, out_ref):
    dinv = dinv_ref[...]
    agg = ((ys_ref[0:N, :] + a_ref[0, 0:N, :] + a_ref[1, 0:N, :])
           * dinv[0:N, None])
    out_ref[...] = agg[:, 0:C] + b_ref[...]


def _tc_fin(ys, a, dinv, b):
    return pl.pallas_call(
        _tc_fin_body,
        out_shape=jax.ShapeDtypeStruct((N, C), _F32),
    )(ys, a, dinv, b)


def kernel(x, edge_index, W1, b1, g1, be1, W2, b2, g2, be2, W3, b3):
    src = edge_index[0]
    dst = edge_index[1]
    fill = jnp.asarray(N + (np.arange(EP - E) % (NP - N)), jnp.int32)
    src_p = jnp.concatenate([src, fill]).reshape(NT, CPT, CH)
    dst_p = jnp.concatenate([dst, fill]).reshape(NT, CPT, CH)
    z128 = jnp.zeros((CH, H), _F32)
    z16 = jnp.zeros((CH, 16), _F32)
    W3p = jnp.pad(W3, ((0, 0), (0, 16 - C)))

    deg_call, agg128_call, agg16_call = _sc_calls()
    hw1 = _tc_mm(x, W1)
    d0, d1 = deg_call(dst_p)
    y1, dinv = _tc_prep(hw1, d0, d1)
    a = agg128_call(y1, z128, src_p, dst_p)
    y2 = _tc_mid(y1, a, dinv, b1, g1, be1, W2)
    a = agg128_call(y2, z128, src_p, dst_p)
    y3 = _tc_last(y2, a, dinv, b2, g2, be2, W3p)
    a = agg16_call(y3, z16, src_p, dst_p)
    return _tc_fin(y3, a, dinv, b3)


# speedup vs baseline: 1.0750x; 1.0750x over previous
import functools

import jax
import jax.numpy as jnp
import numpy as np
from jax import lax
from jax.experimental import pallas as pl
from jax.experimental.pallas import tpu as pltpu
from jax.experimental.pallas import tpu_sc as plsc

N = 10000
D = 128
H = 128
C = 2
E = 320000

NC = 2
NS = 16
NT = NC * NS
CH = 128
CPT = 80
BLK = 40
EP = NT * CPT * CH
NP = 10240
RPT = NP // NS
RPB = RPT // CH

_F32 = jnp.float32


def _sc_mesh():
    return plsc.VectorSubcoreMesh(
        core_axis_name="c", subcore_axis_name="s",
        num_cores=NC, num_subcores=NS)


def _make_deg():
    @functools.partial(
        pl.kernel,
        out_type=[jax.ShapeDtypeStruct((NP,), _F32),
                  jax.ShapeDtypeStruct((NP,), _F32)],
        mesh=_sc_mesh(),
        scratch_types=[
            pltpu.VMEM((CPT, CH), jnp.int32),
            pltpu.VMEM((CH,), _F32),
            pltpu.VMEM((RPT,), _F32),
            pltpu.VMEM_SHARED((NP,), _F32),
        ],
    )
    def deg_kernel(dst_hbm, out0_hbm, out1_hbm, dst_v, ones_v, deg_v, acc):
        cid = lax.axis_index("c")
        sid = lax.axis_index("s")
        wid = cid * NS + sid
        r0 = sid * RPT

        for k in range(CH // 16):
            ones_v[pl.ds(k * 16, 16)] = jnp.ones((16,), _F32)

        def zb(i, carry):
            deg_v[pl.ds(i * 16, 16)] = jnp.zeros((16,), _F32)
            return carry

        lax.fori_loop(0, RPT // 16, zb, 0)
        pltpu.sync_copy(deg_v, acc.at[pl.ds(r0, RPT)])
        pltpu.sync_copy(dst_hbm.at[wid], dst_v)
        plsc.subcore_barrier()

        def body(j, carry):
            pltpu.sync_copy(ones_v, acc.at[dst_v.at[j]], add=True)
            return carry

        lax.fori_loop(0, CPT, body, 0)
        plsc.subcore_barrier()
        pltpu.sync_copy(acc.at[pl.ds(r0, RPT)], deg_v)

        @pl.when(cid == 0)
        def _():
            pltpu.sync_copy(deg_v, out0_hbm.at[pl.ds(r0, RPT)])

        @pl.when(cid != 0)
        def _():
            pltpu.sync_copy(deg_v, out1_hbm.at[pl.ds(r0, RPT)])

    return deg_kernel


def _make_agg(F, tc_tiling=True):
    @functools.partial(
        pl.kernel,
        out_type=jax.ShapeDtypeStruct((NC, NP, F), _F32),
        mesh=_sc_mesh(),
        compiler_params=(None if tc_tiling else
                         pltpu.CompilerParams(use_tc_tiling_on_sc=False)),
        scratch_types=[
            pltpu.VMEM((BLK, CH), jnp.int32),
            pltpu.VMEM((BLK, CH), jnp.int32),
            pltpu.VMEM((CH, F), _F32),
            pltpu.VMEM((CH, F), _F32),
            pltpu.VMEM_SHARED((NP, F), _F32),
            pltpu.SemaphoreType.DMA,
            pltpu.SemaphoreType.DMA,
        ],
    )
    def agg_kernel(y_hbm, zeros_hbm, src_hbm, dst_hbm, out_hbm,
                   src_v, dst_v, buf_a, buf_b, acc, sem_a, sem_b):
        cid = lax.axis_index("c")
        sid = lax.axis_index("s")
        wid = cid * NS + sid
        r0 = sid * RPT

        pltpu.sync_copy(zeros_hbm, buf_a)

        def ib(i, carry):
            pltpu.sync_copy(buf_a, acc.at[pl.ds(r0 + i * CH, CH)])
            return carry

        lax.fori_loop(0, RPB, ib, 0)
        plsc.subcore_barrier()

        def blk(b, carry):
            pltpu.sync_copy(src_hbm.at[wid, pl.ds(b * BLK, BLK)], src_v)
            pltpu.sync_copy(dst_hbm.at[wid, pl.ds(b * BLK, BLK)], dst_v)
            pltpu.async_copy(y_hbm.at[src_v.at[0]], buf_a, sem_a)

            def body(j, carry2):
                ca = 2 * j
                cb = 2 * j + 1
                pltpu.async_copy(y_hbm.at[src_v.at[cb]], buf_b, sem_b)
                pltpu.make_async_copy(
                    y_hbm.at[src_v.at[ca]], buf_a, sem_a).wait()
                pltpu.sync_copy(buf_a, acc.at[dst_v.at[ca]], add=True)
                nxt = lax.rem(ca + 2, BLK)
                pltpu.async_copy(y_hbm.at[src_v.at[nxt]], buf_a, sem_a)
                pltpu.make_async_copy(
                    y_hbm.at[src_v.at[cb]], buf_b, sem_b).wait()
                pltpu.sync_copy(buf_b, acc.at[dst_v.at[cb]], add=True)
                return carry2

            lax.fori_loop(0, BLK // 2, body, 0)
            pltpu.make_async_copy(y_hbm.at[src_v.at[0]], buf_a, sem_a).wait()
            return carry

        lax.fori_loop(0, CPT // BLK, blk, 0)
        plsc.subcore_barrier()

        def ob(i, carry):
            pltpu.sync_copy(acc.at[pl.ds(r0 + i * CH, CH)], buf_a)
            pltpu.sync_copy(buf_a, out_hbm.at[cid, pl.ds(r0 + i * CH, CH)])
            return carry

        lax.fori_loop(0, RPB, ob, 0)

    return agg_kernel


@functools.cache
def _sc_calls():
    return _make_deg(), _make_agg(H), _make_agg(16, tc_tiling=False)


def _tc_prep_body(x_ref, w_ref, d0_ref, d1_ref, y_ref, dinv_ref):
    deg = d0_ref[...] + d1_ref[...] + 1.0
    dinv = lax.rsqrt(deg)
    dinv_ref[...] = dinv
    hw = jnp.dot(x_ref[...], w_ref[...], preferred_element_type=_F32)
    y_ref[0:N, :] = hw * dinv[0:N, None]
    y_ref[N:NP, :] = jnp.zeros((NP - N, H), _F32)


def _tc_prep(x, w1, d0, d1):
    return pl.pallas_call(
        _tc_prep_body,
        out_shape=[
            jax.ShapeDtypeStruct((NP, H), _F32),
            jax.ShapeDtypeStruct((NP,), _F32),
        ],
    )(x, w1, d0, d1)


def _tc_mid_body(ys_ref, a_ref, dinv_ref, b_ref, g_ref, be_ref,
                 w_ref, y_ref):
    dinv = dinv_ref[...]
    agg = ys_ref[0:N, :] + a_ref[0, 0:N, :] + a_ref[1, 0:N, :]
    t = (agg * dinv[0:N, None]) + b_ref[...]
    mu = jnp.mean(t, axis=0)
    var = jnp.mean((t - mu) ** 2, axis=0)
    hbn = (t - mu) * lax.rsqrt(var + 1e-5) * g_ref[...] + be_ref[...]
    hbn = jnp.maximum(hbn, 0.0)
    hw = jnp.dot(hbn, w_ref[...], preferred_element_type=_F32)
    y_ref[0:N, :] = hw * dinv[0:N, None]
    y_ref[N:NP, :] = jnp.zeros((NP - N, w_ref.shape[1]), _F32)


def _tc_mid(ys, a, dinv, b, g, be, w):
    fout = w.shape[1]
    return pl.pallas_call(
        _tc_mid_body,
        out_shape=jax.ShapeDtypeStruct((NP, fout), _F32),
    )(ys, a, dinv, b, g, be, w)


def _tc_last_body(ys_ref, a_ref, dinv_ref, b_ref, g_ref, be_ref, w_ref,
                  y_ref):
    dinv = dinv_ref[...]
    agg = ys_ref[0:N, :] + a_ref[0, 0:N, :] + a_ref[1, 0:N, :]
    t = (agg * dinv[0:N, None]) + b_ref[...]
    mu = jnp.mean(t, axis=0)
    var = jnp.mean((t - mu) ** 2, axis=0)
    hbn = (t - mu) * lax.rsqrt(var + 1e-5) * g_ref[...] + be_ref[...]
    hbn = jnp.maximum(hbn, 0.0)
    hw = jnp.dot(hbn, w_ref[...], preferred_element_type=_F32)
    y_ref[0:N, :] = hw * dinv[0:N, None]
    y_ref[N:NP, :] = jnp.zeros((NP - N, 16), _F32)


def _tc_last(ys, a, dinv, b, g, be, w):
    return pl.pallas_call(
        _tc_last_body,
        out_shape=jax.ShapeDtypeStruct((NP, 16), _F32),
    )(ys, a, dinv, b, g, be, w)


def _tc_fin_body(ys_ref, a_ref, dinv_ref, b_ref, out_ref):
    dinv = dinv_ref[...]
    agg = ((ys_ref[0:N, :] + a_ref[0, 0:N, :] + a_ref[1, 0:N, :])
           * dinv[0:N, None])
    out_ref[...] = agg[:, 0:C] + b_ref[...]


def _tc_fin(ys, a, dinv, b):
    return pl.pallas_call(
        _tc_fin_body,
        out_shape=jax.ShapeDtypeStruct((N, C), _F32),
    )(ys, a, dinv, b)


def kernel(x, edge_index, W1, b1, g1, be1, W2, b2, g2, be2, W3, b3):
    src = edge_index[0]
    dst = edge_index[1]
    fill = jnp.asarray(N + (np.arange(EP - E) % (NP - N)), jnp.int32)
    src_p = jnp.concatenate([src, fill]).reshape(NT, CPT, CH)
    dst_p = jnp.concatenate([dst, fill]).reshape(NT, CPT, CH)
    z128 = jnp.zeros((CH, H), _F32)
    z16 = jnp.zeros((CH, 16), _F32)
    W3p = jnp.pad(W3, ((0, 0), (0, 16 - C)))

    deg_call, agg128_call, agg16_call = _sc_calls()
    d0, d1 = deg_call(dst_p)
    y1, dinv = _tc_prep(x, W1, d0, d1)
    a = agg128_call(y1, z128, src_p, dst_p)
    y2 = _tc_mid(y1, a, dinv, b1, g1, be1, W2)
    a = agg128_call(y2, z128, src_p, dst_p)
    y3 = _tc_last(y2, a, dinv, b2, g2, be2, W3p)
    a = agg16_call(y3, z16, src_p, dst_p)
    return _tc_fin(y3, a, dinv, b3)


# speedup vs baseline: 1.1149x; 1.0371x over previous
import functools

import jax
import jax.numpy as jnp
import numpy as np
from jax import lax
from jax.experimental import pallas as pl
from jax.experimental.pallas import tpu as pltpu
from jax.experimental.pallas import tpu_sc as plsc

N = 10000
D = 128
H = 128
C = 2
E = 320000

NC = 2
NS = 16
NT = NC * NS
CH = 128
CPT = 80
BLK = 40
EP = NT * CPT * CH
NP = 10240
RPT = NP // NS
RPB = RPT // CH

_F32 = jnp.float32


def _sc_mesh():
    return plsc.VectorSubcoreMesh(
        core_axis_name="c", subcore_axis_name="s",
        num_cores=NC, num_subcores=NS)


def _make_deg():
    @functools.partial(
        pl.kernel,
        out_type=[jax.ShapeDtypeStruct((NP,), _F32),
                  jax.ShapeDtypeStruct((NP,), _F32)],
        mesh=_sc_mesh(),
        scratch_types=[
            pltpu.VMEM((CPT, CH), jnp.int32),
            pltpu.VMEM((CH,), _F32),
            pltpu.VMEM((RPT,), _F32),
            pltpu.VMEM_SHARED((NP,), _F32),
        ],
    )
    def deg_kernel(dst_hbm, out0_hbm, out1_hbm, dst_v, ones_v, deg_v, acc):
        cid = lax.axis_index("c")
        sid = lax.axis_index("s")
        wid = cid * NS + sid
        r0 = sid * RPT

        for k in range(CH // 16):
            ones_v[pl.ds(k * 16, 16)] = jnp.ones((16,), _F32)

        def zb(i, carry):
            deg_v[pl.ds(i * 16, 16)] = jnp.zeros((16,), _F32)
            return carry

        lax.fori_loop(0, RPT // 16, zb, 0)
        pltpu.sync_copy(deg_v, acc.at[pl.ds(r0, RPT)])
        pltpu.sync_copy(dst_hbm.at[wid], dst_v)
        plsc.subcore_barrier()

        def body(j, carry):
            pltpu.sync_copy(ones_v, acc.at[dst_v.at[j]], add=True)
            return carry

        lax.fori_loop(0, CPT, body, 0)
        plsc.subcore_barrier()
        pltpu.sync_copy(acc.at[pl.ds(r0, RPT)], deg_v)

        @pl.when(cid == 0)
        def _():
            pltpu.sync_copy(deg_v, out0_hbm.at[pl.ds(r0, RPT)])

        @pl.when(cid != 0)
        def _():
            pltpu.sync_copy(deg_v, out1_hbm.at[pl.ds(r0, RPT)])

    return deg_kernel


def _make_agg(F, tc_tiling=True):
    @functools.partial(
        pl.kernel,
        out_type=jax.ShapeDtypeStruct((NC, NP, F), _F32),
        mesh=_sc_mesh(),
        compiler_params=(None if tc_tiling else
                         pltpu.CompilerParams(use_tc_tiling_on_sc=False)),
        scratch_types=[
            pltpu.VMEM((BLK, CH), jnp.int32),
            pltpu.VMEM((BLK, CH), jnp.int32),
            pltpu.VMEM((CH, F), _F32),
            pltpu.VMEM((CH, F), _F32),
            pltpu.VMEM_SHARED((NP, F), _F32),
            pltpu.SemaphoreType.DMA,
            pltpu.SemaphoreType.DMA,
        ],
    )
    def agg_kernel(y_hbm, zeros_hbm, src_hbm, dst_hbm, out_hbm,
                   src_v, dst_v, buf_a, buf_b, acc, sem_a, sem_b):
        cid = lax.axis_index("c")
        sid = lax.axis_index("s")
        wid = cid * NS + sid
        r0 = sid * RPT

        pltpu.sync_copy(zeros_hbm, buf_a)

        def ib(i, carry):
            pltpu.sync_copy(buf_a, acc.at[pl.ds(r0 + i * CH, CH)])
            return carry

        lax.fori_loop(0, RPB, ib, 0)
        plsc.subcore_barrier()

        def blk(b, carry):
            pltpu.sync_copy(src_hbm.at[wid, pl.ds(b * BLK, BLK)], src_v)
            pltpu.sync_copy(dst_hbm.at[wid, pl.ds(b * BLK, BLK)], dst_v)
            pltpu.async_copy(y_hbm.at[src_v.at[0]], buf_a, sem_a)

            def body(j, carry2):
                ca = 2 * j
                cb = 2 * j + 1
                pltpu.async_copy(y_hbm.at[src_v.at[cb]], buf_b, sem_b)
                pltpu.make_async_copy(
                    y_hbm.at[src_v.at[ca]], buf_a, sem_a).wait()
                pltpu.sync_copy(buf_a, acc.at[dst_v.at[ca]], add=True)
                nxt = lax.rem(ca + 2, BLK)
                pltpu.async_copy(y_hbm.at[src_v.at[nxt]], buf_a, sem_a)
                pltpu.make_async_copy(
                    y_hbm.at[src_v.at[cb]], buf_b, sem_b).wait()
                pltpu.sync_copy(buf_b, acc.at[dst_v.at[cb]], add=True)
                return carry2

            lax.fori_loop(0, BLK // 2, body, 0)
            pltpu.make_async_copy(y_hbm.at[src_v.at[0]], buf_a, sem_a).wait()
            return carry

        lax.fori_loop(0, CPT // BLK, blk, 0)
        plsc.subcore_barrier()

        def ob(i, carry):
            pltpu.sync_copy(acc.at[pl.ds(r0 + i * CH, CH)], buf_a)
            pltpu.sync_copy(buf_a, out_hbm.at[cid, pl.ds(r0 + i * CH, CH)])
            return carry

        lax.fori_loop(0, RPB, ob, 0)

    return agg_kernel


def _make_agg16():
    F = 16
    DEEP = 4
    LAG = 2

    @functools.partial(
        pl.kernel,
        out_type=jax.ShapeDtypeStruct((NC, NP, F), _F32),
        mesh=_sc_mesh(),
        compiler_params=pltpu.CompilerParams(use_tc_tiling_on_sc=False),
        scratch_types=[
            pltpu.VMEM((BLK, CH), jnp.int32),
            pltpu.VMEM((BLK, CH), jnp.int32),
            [pltpu.VMEM((CH, F), _F32) for _ in range(DEEP)],
            pltpu.VMEM_SHARED((NP, F), _F32),
            [pltpu.SemaphoreType.DMA for _ in range(DEEP)],
            [pltpu.SemaphoreType.DMA for _ in range(DEEP)],
        ],
    )
    def agg16_kernel(y_hbm, zeros_hbm, src_hbm, dst_hbm, out_hbm,
                     src_v, dst_v, bufs, acc, sem_g, sem_s):
        cid = lax.axis_index("c")
        sid = lax.axis_index("s")
        wid = cid * NS + sid
        r0 = sid * RPT

        pltpu.sync_copy(zeros_hbm, bufs[0])

        def ib(i, carry):
            pltpu.sync_copy(bufs[0], acc.at[pl.ds(r0 + i * CH, CH)])
            return carry

        lax.fori_loop(0, RPB, ib, 0)
        plsc.subcore_barrier()

        def blk(b, carry):
            pltpu.sync_copy(src_hbm.at[wid, pl.ds(b * BLK, BLK)], src_v)
            pltpu.sync_copy(dst_hbm.at[wid, pl.ds(b * BLK, BLK)], dst_v)
            for m in range(DEEP):
                pltpu.async_copy(y_hbm.at[src_v.at[m]], bufs[m], sem_g[m])

            def body(j, carry2):
                for m in range(DEEP):
                    c = DEEP * j + m
                    mw = (m + LAG) % DEEP

                    def refill():
                        pltpu.make_async_copy(
                            bufs[mw], acc.at[dst_v.at[0]], sem_s[mw]).wait()
                        pltpu.async_copy(
                            y_hbm.at[src_v.at[lax.rem(c + LAG, BLK)]],
                            bufs[mw], sem_g[mw])

                    if m >= LAG:
                        refill()
                    else:
                        pl.when(j >= 1)(refill)
                    pltpu.make_async_copy(
                        y_hbm.at[src_v.at[c]], bufs[m], sem_g[m]).wait()
                    pltpu.async_copy(bufs[m], acc.at[dst_v.at[c]],
                                     sem_s[m], add=True)
                return carry2

            lax.fori_loop(0, BLK // DEEP, body, 0)
            for m in range(LAG):
                pltpu.make_async_copy(
                    y_hbm.at[src_v.at[0]], bufs[m], sem_g[m]).wait()
            for m in range(LAG, DEEP):
                pltpu.make_async_copy(
                    bufs[m], acc.at[dst_v.at[0]], sem_s[m]).wait()
            return carry

        lax.fori_loop(0, CPT // BLK, blk, 0)
        plsc.subcore_barrier()

        def ob(i, carry):
            pltpu.sync_copy(acc.at[pl.ds(r0 + i * CH, CH)], bufs[0])
            pltpu.sync_copy(bufs[0], out_hbm.at[cid, pl.ds(r0 + i * CH, CH)])
            return carry

        lax.fori_loop(0, RPB, ob, 0)

    return agg16_kernel


@functools.cache
def _sc_calls():
    return _make_deg(), _make_agg(H), _make_agg16()


def _tc_prep_body(x_ref, w_ref, d0_ref, d1_ref, y_ref, dinv_ref):
    deg = d0_ref[...] + d1_ref[...] + 1.0
    dinv = lax.rsqrt(deg)
    dinv_ref[...] = dinv
    hw = jnp.dot(x_ref[...], w_ref[...], preferred_element_type=_F32)
    y_ref[0:N, :] = hw * dinv[0:N, None]
    y_ref[N:NP, :] = jnp.zeros((NP - N, H), _F32)


def _tc_prep(x, w1, d0, d1):
    return pl.pallas_call(
        _tc_prep_body,
        out_shape=[
            jax.ShapeDtypeStruct((NP, H), _F32),
            jax.ShapeDtypeStruct((NP,), _F32),
        ],
    )(x, w1, d0, d1)


def _tc_mid_body(ys_ref, a_ref, dinv_ref, b_ref, g_ref, be_ref,
                 w_ref, y_ref):
    dinv = dinv_ref[...]
    agg = ys_ref[0:N, :] + a_ref[0, 0:N, :] + a_ref[1, 0:N, :]
    t = (agg * dinv[0:N, None]) + b_ref[...]
    mu = jnp.mean(t, axis=0)
    var = jnp.mean((t - mu) ** 2, axis=0)
    hbn = (t - mu) * lax.rsqrt(var + 1e-5) * g_ref[...] + be_ref[...]
    hbn = jnp.maximum(hbn, 0.0)
    hw = jnp.dot(hbn, w_ref[...], preferred_element_type=_F32)
    y_ref[0:N, :] = hw * dinv[0:N, None]
    y_ref[N:NP, :] = jnp.zeros((NP - N, w_ref.shape[1]), _F32)


def _tc_mid(ys, a, dinv, b, g, be, w):
    fout = w.shape[1]
    return pl.pallas_call(
        _tc_mid_body,
        out_shape=jax.ShapeDtypeStruct((NP, fout), _F32),
    )(ys, a, dinv, b, g, be, w)


def _tc_last_body(ys_ref, a_ref, dinv_ref, b_ref, g_ref, be_ref, w_ref,
                  y_ref):
    dinv = dinv_ref[...]
    agg = ys_ref[0:N, :] + a_ref[0, 0:N, :] + a_ref[1, 0:N, :]
    t = (agg * dinv[0:N, None]) + b_ref[...]
    mu = jnp.mean(t, axis=0)
    var = jnp.mean((t - mu) ** 2, axis=0)
    hbn = (t - mu) * lax.rsqrt(var + 1e-5) * g_ref[...] + be_ref[...]
    hbn = jnp.maximum(hbn, 0.0)
    hw = jnp.dot(hbn, w_ref[...], preferred_element_type=_F32)
    y_ref[0:N, :] = hw * dinv[0:N, None]
    y_ref[N:NP, :] = jnp.zeros((NP - N, 16), _F32)


def _tc_last(ys, a, dinv, b, g, be, w):
    return pl.pallas_call(
        _tc_last_body,
        out_shape=jax.ShapeDtypeStruct((NP, 16), _F32),
    )(ys, a, dinv, b, g, be, w)


def _tc_fin_body(ys_ref, a_ref, dinv_ref, b_ref, out_ref):
    dinv = dinv_ref[...]
    agg = ((ys_ref[0:N, :] + a_ref[0, 0:N, :] + a_ref[1, 0:N, :])
           * dinv[0:N, None])
    out_ref[...] = agg[:, 0:C] + b_ref[...]


def _tc_fin(ys, a, dinv, b):
    return pl.pallas_call(
        _tc_fin_body,
        out_shape=jax.ShapeDtypeStruct((N, C), _F32),
    )(ys, a, dinv, b)


def kernel(x, edge_index, W1, b1, g1, be1, W2, b2, g2, be2, W3, b3):
    src = edge_index[0]
    dst = edge_index[1]
    fill = jnp.asarray(N + (np.arange(EP - E) % (NP - N)), jnp.int32)
    src_p = jnp.concatenate([src, fill]).reshape(NT, CPT, CH)
    dst_p = jnp.concatenate([dst, fill]).reshape(NT, CPT, CH)
    z128 = jnp.zeros((CH, H), _F32)
    z16 = jnp.zeros((CH, 16), _F32)
    W3p = jnp.pad(W3, ((0, 0), (0, 16 - C)))

    deg_call, agg128_call, agg16_call = _sc_calls()
    d0, d1 = deg_call(dst_p)
    y1, dinv = _tc_prep(x, W1, d0, d1)
    a = agg128_call(y1, z128, src_p, dst_p)
    y2 = _tc_mid(y1, a, dinv, b1, g1, be1, W2)
    a = agg128_call(y2, z128, src_p, dst_p)
    y3 = _tc_last(y2, a, dinv, b2, g2, be2, W3p)
    a = agg16_call(y3, z16, src_p, dst_p)
    return _tc_fin(y3, a, dinv, b3)


# speedup vs baseline: 1.1166x; 1.0016x over previous
import functools

import jax
import jax.numpy as jnp
import numpy as np
from jax import lax
from jax.experimental import pallas as pl
from jax.experimental.pallas import tpu as pltpu
from jax.experimental.pallas import tpu_sc as plsc

N = 10000
D = 128
H = 128
C = 2
E = 320000

NC = 2
NS = 16
NT = NC * NS
CH = 128
CPT = 80
BLK = 40
EP = NT * CPT * CH
NP = 10240
RPT = NP // NS
RPB = RPT // CH

_F32 = jnp.float32


def _sc_mesh():
    return plsc.VectorSubcoreMesh(
        core_axis_name="c", subcore_axis_name="s",
        num_cores=NC, num_subcores=NS)


def _make_deg():
    @functools.partial(
        pl.kernel,
        out_type=[jax.ShapeDtypeStruct((NP,), _F32),
                  jax.ShapeDtypeStruct((NP,), _F32)],
        mesh=_sc_mesh(),
        scratch_types=[
            pltpu.VMEM((CPT, CH), jnp.int32),
            pltpu.VMEM((CH,), _F32),
            pltpu.VMEM((RPT,), _F32),
            pltpu.VMEM_SHARED((NP,), _F32),
        ],
    )
    def deg_kernel(dst_hbm, out0_hbm, out1_hbm, dst_v, ones_v, deg_v, acc):
        cid = lax.axis_index("c")
        sid = lax.axis_index("s")
        wid = cid * NS + sid
        r0 = sid * RPT

        for k in range(CH // 16):
            ones_v[pl.ds(k * 16, 16)] = jnp.ones((16,), _F32)

        def zb(i, carry):
            deg_v[pl.ds(i * 16, 16)] = jnp.zeros((16,), _F32)
            return carry

        lax.fori_loop(0, RPT // 16, zb, 0)
        pltpu.sync_copy(deg_v, acc.at[pl.ds(r0, RPT)])
        pltpu.sync_copy(dst_hbm.at[wid], dst_v)
        plsc.subcore_barrier()

        def body(j, carry):
            pltpu.sync_copy(ones_v, acc.at[dst_v.at[j]], add=True)
            return carry

        lax.fori_loop(0, CPT, body, 0)
        plsc.subcore_barrier()
        pltpu.sync_copy(acc.at[pl.ds(r0, RPT)], deg_v)

        @pl.when(cid == 0)
        def _():
            pltpu.sync_copy(deg_v, out0_hbm.at[pl.ds(r0, RPT)])

        @pl.when(cid != 0)
        def _():
            pltpu.sync_copy(deg_v, out1_hbm.at[pl.ds(r0, RPT)])

    return deg_kernel


def _make_agg(F):
    @functools.partial(
        pl.kernel,
        out_type=jax.ShapeDtypeStruct((NC, NP, F), _F32),
        mesh=_sc_mesh(),
        scratch_types=[
            pltpu.VMEM((BLK, CH), jnp.int32),
            pltpu.VMEM((BLK, CH), jnp.int32),
            pltpu.VMEM((CH, F), _F32),
            pltpu.VMEM((CH, F), _F32),
            pltpu.VMEM_SHARED((NP, F), _F32),
            pltpu.SemaphoreType.DMA,
            pltpu.SemaphoreType.DMA,
        ],
    )
    def agg_kernel(y_hbm, zeros_hbm, src_hbm, dst_hbm, out_hbm,
                   src_v, dst_v, buf_a, buf_b, acc, sem_a, sem_b):
        cid = lax.axis_index("c")
        sid = lax.axis_index("s")
        wid = cid * NS + sid
        r0 = sid * RPT

        pltpu.sync_copy(zeros_hbm, buf_a)

        def ib(i, carry):
            pltpu.sync_copy(buf_a, acc.at[pl.ds(r0 + i * CH, CH)])
            return carry

        lax.fori_loop(0, RPB, ib, 0)
        plsc.subcore_barrier()

        def blk(b, carry):
            pltpu.sync_copy(src_hbm.at[wid, pl.ds(b * BLK, BLK)], src_v)
            pltpu.sync_copy(dst_hbm.at[wid, pl.ds(b * BLK, BLK)], dst_v)
            pltpu.async_copy(y_hbm.at[src_v.at[0]], buf_a, sem_a)

            def body(j, carry2):
                ca = 2 * j
                cb = 2 * j + 1
                pltpu.async_copy(y_hbm.at[src_v.at[cb]], buf_b, sem_b)
                pltpu.make_async_copy(
                    y_hbm.at[src_v.at[ca]], buf_a, sem_a).wait()
                pltpu.sync_copy(buf_a, acc.at[dst_v.at[ca]], add=True)
                nxt = lax.rem(ca + 2, BLK)
                pltpu.async_copy(y_hbm.at[src_v.at[nxt]], buf_a, sem_a)
                pltpu.make_async_copy(
                    y_hbm.at[src_v.at[cb]], buf_b, sem_b).wait()
                pltpu.sync_copy(buf_b, acc.at[dst_v.at[cb]], add=True)
                return carry2

            lax.fori_loop(0, BLK // 2, body, 0)
            pltpu.make_async_copy(y_hbm.at[src_v.at[0]], buf_a, sem_a).wait()
            return carry

        lax.fori_loop(0, CPT // BLK, blk, 0)
        plsc.subcore_barrier()

        def ob(i, carry):
            pltpu.sync_copy(acc.at[pl.ds(r0 + i * CH, CH)], buf_a)
            pltpu.sync_copy(buf_a, out_hbm.at[cid, pl.ds(r0 + i * CH, CH)])
            return carry

        lax.fori_loop(0, RPB, ob, 0)

    return agg_kernel


def _make_agg16():
    F = 16
    DEEP = 4
    LAG = 2

    @functools.partial(
        pl.kernel,
        out_type=jax.ShapeDtypeStruct((NC, NP, F), _F32),
        mesh=_sc_mesh(),
        compiler_params=pltpu.CompilerParams(use_tc_tiling_on_sc=False),
        scratch_types=[
            pltpu.VMEM((BLK, CH), jnp.int32),
            pltpu.VMEM((BLK, CH), jnp.int32),
            [pltpu.VMEM((CH, F), _F32) for _ in range(DEEP)],
            pltpu.VMEM_SHARED((NP, F), _F32),
            [pltpu.SemaphoreType.DMA for _ in range(DEEP)],
            [pltpu.SemaphoreType.DMA for _ in range(DEEP)],
        ],
    )
    def agg16_kernel(y_hbm, zeros_hbm, src_hbm, dst_hbm, out_hbm,
                     src_v, dst_v, bufs, acc, sem_g, sem_s):
        cid = lax.axis_index("c")
        sid = lax.axis_index("s")
        wid = cid * NS + sid
        r0 = sid * RPT

        pltpu.sync_copy(zeros_hbm, bufs[0])

        def ib(i, carry):
            pltpu.sync_copy(bufs[0], acc.at[pl.ds(r0 + i * CH, CH)])
            return carry

        lax.fori_loop(0, RPB, ib, 0)
        plsc.subcore_barrier()

        def blk(b, carry):
            pltpu.sync_copy(src_hbm.at[wid, pl.ds(b * BLK, BLK)], src_v)
            pltpu.sync_copy(dst_hbm.at[wid, pl.ds(b * BLK, BLK)], dst_v)
            for m in range(DEEP):
                pltpu.async_copy(y_hbm.at[src_v.at[m]], bufs[m], sem_g[m])

            def body(j, carry2):
                for m in range(DEEP):
                    c = DEEP * j + m
                    mw = (m + LAG) % DEEP

                    def refill():
                        pltpu.make_async_copy(
                            bufs[mw], acc.at[dst_v.at[0]], sem_s[mw]).wait()
                        pltpu.async_copy(
                            y_hbm.at[src_v.at[lax.rem(c + LAG, BLK)]],
                            bufs[mw], sem_g[mw])

                    if m >= LAG:
                        refill()
                    else:
                        pl.when(j >= 1)(refill)
                    pltpu.make_async_copy(
                        y_hbm.at[src_v.at[c]], bufs[m], sem_g[m]).wait()
                    pltpu.async_copy(bufs[m], acc.at[dst_v.at[c]],
                                     sem_s[m], add=True)
                return carry2

            lax.fori_loop(0, BLK // DEEP, body, 0)
            for m in range(LAG):
                pltpu.make_async_copy(
                    y_hbm.at[src_v.at[0]], bufs[m], sem_g[m]).wait()
            for m in range(LAG, DEEP):
                pltpu.make_async_copy(
                    bufs[m], acc.at[dst_v.at[0]], sem_s[m]).wait()
            return carry

        lax.fori_loop(0, CPT // BLK, blk, 0)
        plsc.subcore_barrier()

        def ob(i, carry):
            pltpu.sync_copy(acc.at[pl.ds(r0 + i * CH, CH)], bufs[0])
            pltpu.sync_copy(bufs[0], out_hbm.at[cid, pl.ds(r0 + i * CH, CH)])
            return carry

        lax.fori_loop(0, RPB, ob, 0)

    return agg16_kernel


@functools.cache
def _sc_calls():
    return _make_deg(), _make_agg(H), _make_agg16()


def _tc_prep_body(x_ref, w_ref, d0_ref, d1_ref, y_ref, dinv_ref):
    deg = d0_ref[...] + d1_ref[...] + 1.0
    dinv = lax.rsqrt(deg)
    dinv_ref[...] = dinv
    hw = jnp.dot(x_ref[...], w_ref[...], preferred_element_type=_F32)
    y_ref[0:N, :] = hw * dinv[0:N, None]
    y_ref[N:NP, :] = jnp.zeros((NP - N, H), _F32)


def _tc_prep(x, w1, d0, d1):
    return pl.pallas_call(
        _tc_prep_body,
        out_shape=[
            jax.ShapeDtypeStruct((NP, H), _F32),
            jax.ShapeDtypeStruct((NP,), _F32),
        ],
    )(x, w1, d0, d1)


def _tc_mid_body(ys_ref, a_ref, dinv_ref, b_ref, g_ref, be_ref,
                 w_ref, y_ref):
    dinv = dinv_ref[...]
    agg = ys_ref[0:N, :] + a_ref[0, 0:N, :] + a_ref[1, 0:N, :]
    t = (agg * dinv[0:N, None]) + b_ref[...]
    mu = jnp.mean(t, axis=0)
    var = jnp.mean((t - mu) ** 2, axis=0)
    hbn = (t - mu) * lax.rsqrt(var + 1e-5) * g_ref[...] + be_ref[...]
    hbn = jnp.maximum(hbn, 0.0)
    hw = jnp.dot(hbn, w_ref[...], preferred_element_type=_F32)
    y_ref[0:N, :] = hw * dinv[0:N, None]
    y_ref[N:NP, :] = jnp.zeros((NP - N, w_ref.shape[1]), _F32)


def _tc_mid(ys, a, dinv, b, g, be, w):
    fout = w.shape[1]
    return pl.pallas_call(
        _tc_mid_body,
        out_shape=jax.ShapeDtypeStruct((NP, fout), _F32),
    )(ys, a, dinv, b, g, be, w)


def _tc_last_body(ys_ref, a_ref, dinv_ref, b_ref, g_ref, be_ref, w_ref,
                  y_ref):
    dinv = dinv_ref[...]
    agg = ys_ref[0:N, :] + a_ref[0, 0:N, :] + a_ref[1, 0:N, :]
    t = (agg * dinv[0:N, None]) + b_ref[...]
    mu = jnp.mean(t, axis=0)
    var = jnp.mean((t - mu) ** 2, axis=0)
    hbn = (t - mu) * lax.rsqrt(var + 1e-5) * g_ref[...] + be_ref[...]
    hbn = jnp.maximum(hbn, 0.0)
    hw = jnp.dot(hbn, w_ref[...], preferred_element_type=_F32)
    y_ref[0:N, :] = hw * dinv[0:N, None]
    y_ref[N:NP, :] = jnp.zeros((NP - N, 16), _F32)


def _tc_last(ys, a, dinv, b, g, be, w):
    return pl.pallas_call(
        _tc_last_body,
        out_shape=jax.ShapeDtypeStruct((NP, 16), _F32),
    )(ys, a, dinv, b, g, be, w)


def _tc_fin_body(ys_ref, a_ref, dinv_ref, b_ref, out_ref):
    dinv = dinv_ref[...]
    agg = ((ys_ref[0:N, :] + a_ref[0, 0:N, :] + a_ref[1, 0:N, :])
           * dinv[0:N, None])
    out_ref[...] = agg[:, 0:C] + b_ref[...]


def _tc_fin(ys, a, dinv, b):
    return pl.pallas_call(
        _tc_fin_body,
        out_shape=jax.ShapeDtypeStruct((N, C), _F32),
    )(ys, a, dinv, b)


def kernel(x, edge_index, W1, b1, g1, be1, W2, b2, g2, be2, W3, b3):
    src = edge_index[0]
    dst = edge_index[1]
    fill = jnp.asarray(N + (np.arange(EP - E) % (NP - N)), jnp.int32)
    src_p = jnp.concatenate([src, fill]).reshape(NT, CPT, CH)
    dst_p = jnp.concatenate([dst, fill]).reshape(NT, CPT, CH)
    z128 = jnp.zeros((CH, H), _F32)
    z16 = jnp.zeros((CH, 16), _F32)
    W3p = jnp.pad(W3, ((0, 0), (0, 16 - C)))

    deg_call, agg128_call, agg16_call = _sc_calls()
    d0, d1 = deg_call(dst_p)
    y1, dinv = _tc_prep(x, W1, d0, d1)
    a = agg128_call(y1, z128, src_p, dst_p)
    y2 = _tc_mid(y1, a, dinv, b1, g1, be1, W2)
    a = agg128_call(y2, z128, src_p, dst_p)
    y3 = _tc_last(y2, a, dinv, b2, g2, be2, W3p)
    a = agg16_call(y3, z16, src_p, dst_p)
    return _tc_fin(y3, a, dinv, b3)
